# baseline (speedup 1.0000x reference)
import functools

import jax
import jax.numpy as jnp
from jax.experimental import pallas as pl


def _bcast_kernel(pe_ref, out_ref, *, pb):
    i = pl.program_id(0)
    for p in range(pb):
        row = pe_ref[pl.ds(i * pb + p, 1), :]
        col = jnp.swapaxes(row, 0, 1)
        out_ref[p, :, :] = jnp.broadcast_to(col, out_ref.shape[1:])


def kernel(x, pe_weight):
    batch = x.shape[0]
    max_len, d_model = pe_weight.shape
    pb = 20
    out3 = pl.pallas_call(
        functools.partial(_bcast_kernel, pb=pb),
        grid=(max_len // pb,),
        in_specs=[pl.BlockSpec((max_len, d_model), lambda i: (0, 0))],
        out_specs=pl.BlockSpec((pb, d_model, batch), lambda i: (i, 0, 0)),
        out_shape=jax.ShapeDtypeStruct((max_len, d_model, batch), pe_weight.dtype),
    )(pe_weight)
    return out3.transpose(2, 0, 1)


# speedup vs baseline: 1.0712x; 1.0712x over previous
import functools

import jax
import jax.numpy as jnp
from jax.experimental import pallas as pl


def _bcast_kernel(pe_ref, out_ref, *, pb):
    i = pl.program_id(0)
    for p in range(pb):
        row = pe_ref[pl.ds(i * pb + p, 1), :]
        col = jnp.swapaxes(row, 0, 1)
        out_ref[p, :, :] = jnp.broadcast_to(col, out_ref.shape[1:])


def kernel(x, pe_weight):
    batch = x.shape[0]
    max_len, d_model = pe_weight.shape
    pb = 4
    out3 = pl.pallas_call(
        functools.partial(_bcast_kernel, pb=pb),
        grid=(max_len // pb,),
        in_specs=[pl.BlockSpec((max_len, d_model), lambda i: (0, 0))],
        out_specs=pl.BlockSpec((pb, d_model, batch), lambda i: (i, 0, 0)),
        out_shape=jax.ShapeDtypeStruct((max_len, d_model, batch), pe_weight.dtype),
    )(pe_weight)
    return out3.transpose(2, 0, 1)
